# Initial kernel scaffold; baseline (speedup 1.0000x reference)
import jax
import jax.numpy as jnp
from jax.experimental import pallas as pl


def kernel(x, emb_0, emb_1, emb_2, emb_3, emb_4, emb_5, emb_6, emb_7, emb_8, emb_9):
    raise NotImplementedError("write your pallas kernel here")


# speedup vs baseline: 1.3902x; 1.3902x over previous
import functools

import jax
import jax.numpy as jnp
from jax import lax
from jax.experimental import pallas as pl
from jax.experimental.pallas import tpu as pltpu
from jax.experimental.pallas import tpu_sc as plsc

NF = 10
H = 128
NC = 2
NS = 16
NW = NC * NS
C = 256
NSUB = C // 128


def _sc_lookup_sum(n_pad, nchunk):
    mesh = plsc.VectorSubcoreMesh(core_axis_name="c", subcore_axis_name="s")

    @functools.partial(
        pl.kernel,
        out_type=jax.ShapeDtypeStruct((n_pad, H), jnp.float32),
        mesh=mesh,
        scratch_types=[
            pltpu.VMEM((NF, NSUB, 128), jnp.int32),
            pltpu.VMEM((C, H), jnp.float32),
            pltpu.VMEM((2, C, H), jnp.float32),
            pltpu.SemaphoreType.DMA,
            pltpu.SemaphoreType.DMA,
            pltpu.SemaphoreType.DMA,
        ],
    )
    def body(x_hbm, t0, t1, t2, t3, t4, t5, t6, t7, t8, t9, out_hbm,
             idx_v, acc_v, gath_v, sem_a, sem_0, sem_1):
        tabs = [t0, t1, t2, t3, t4, t5, t6, t7, t8, t9]
        sems = [sem_0, sem_1]
        wid = lax.axis_index("s") * NC + lax.axis_index("c")

        def fire(f, k):
            cps = []
            for s in range(NSUB):
                cps.append(pltpu.async_copy(
                    tabs[f].at[idx_v.at[f, s]],
                    gath_v.at[k, pl.ds(s * 128, 128)],
                    sems[k]))
            return cps

        def add_from(k):
            def row_body(r, carry):
                for u in range(2):
                    row = 2 * r + u
                    for v in range(H // 16):
                        plsc.addupdate(
                            acc_v.at[row, pl.ds(16 * v, 16)],
                            gath_v[k, row, pl.ds(16 * v, 16)])
                return carry
            lax.fori_loop(0, C // 2, row_body, 0)

        def chunk_body(j, carry):
            pltpu.sync_copy(x_hbm.at[wid, j], idx_v)
            acc_cps = []
            for s in range(NSUB):
                acc_cps.append(pltpu.async_copy(
                    tabs[0].at[idx_v.at[0, s]],
                    acc_v.at[pl.ds(s * 128, 128)],
                    sem_a))
            prev_cps, prev_buf = fire(1, 0), 0
            for cp in acc_cps:
                cp.wait()
            for f in range(2, NF):
                cur_buf = (f - 1) % 2
                cur_cps = fire(f, cur_buf)
                for cp in prev_cps:
                    cp.wait()
                add_from(prev_buf)
                prev_cps, prev_buf = cur_cps, cur_buf
            for cp in prev_cps:
                cp.wait()
            add_from(prev_buf)
            pltpu.sync_copy(acc_v, out_hbm.at[pl.ds(wid * (nchunk * C) + j * C, C)])
            return carry

        lax.fori_loop(0, nchunk, chunk_body, 0)

    return body


def kernel(x, emb_0, emb_1, emb_2, emb_3, emb_4, emb_5, emb_6, emb_7,
           emb_8, emb_9):
    n = x.shape[0]
    rows_per_w = -(-n // (NW * C)) * C
    nchunk = rows_per_w // C
    n_pad = NW * rows_per_w

    xi = x.reshape(n, NF)
    xi = jnp.pad(xi, ((0, n_pad - n), (0, 0)))
    xb = xi.reshape(NW, nchunk, C, NF).transpose(0, 1, 3, 2)
    xb = xb.reshape(NW, nchunk, NF, NSUB, 128)

    out = _sc_lookup_sum(n_pad, nchunk)(
        xb, emb_0, emb_1, emb_2, emb_3, emb_4, emb_5, emb_6, emb_7,
        emb_8, emb_9)
    return out[:n].reshape(n, 1, H)


# speedup vs baseline: 5.0047x; 3.6001x over previous
import functools

import jax
import jax.numpy as jnp
from jax import lax
from jax.experimental import pallas as pl
from jax.experimental.pallas import tpu as pltpu
from jax.experimental.pallas import tpu_sc as plsc

NF = 10
H = 128
NC = 2
NS = 16
NW = NC * NS
C = 128
NSUB = C // 128


def _sc_lookup_sum(n_pad, nchunk):
    mesh = plsc.VectorSubcoreMesh(core_axis_name="c", subcore_axis_name="s")

    @functools.partial(
        pl.kernel,
        out_type=jax.ShapeDtypeStruct((n_pad, H), jnp.float32),
        mesh=mesh,
        scratch_types=[
            pltpu.VMEM((NF, NSUB, 128), jnp.int32),
            pltpu.VMEM((C, H), jnp.float32),
            pltpu.VMEM((2, C, H), jnp.float32),
            pltpu.VMEM_SHARED((NF * 500, H), jnp.float32),
            pltpu.SemaphoreType.DMA,
            pltpu.SemaphoreType.DMA,
            pltpu.SemaphoreType.DMA,
        ],
    )
    def body(x_hbm, t0, t1, t2, t3, t4, t5, t6, t7, t8, t9, out_hbm,
             idx_v, acc_v, gath_v, sh_tab, sem_a, sem_0, sem_1):
        tabs = [t0, t1, t2, t3, t4, t5, t6, t7, t8, t9]
        sems = [sem_0, sem_1]
        sid = lax.axis_index("s")
        wid = sid * NC + lax.axis_index("c")

        @pl.when(sid == 0)
        def _stage():
            for f in range(NF):
                pltpu.sync_copy(tabs[f], sh_tab.at[pl.ds(500 * f, 500)])
        plsc.subcore_barrier()

        def fire(f, k):
            cps = []
            for s in range(NSUB):
                cps.append(pltpu.async_copy(
                    sh_tab.at[idx_v.at[f, s]],
                    gath_v.at[k, pl.ds(s * 128, 128)],
                    sems[k]))
            return cps

        def add_from(k):
            def row_body(r, carry):
                for u in range(2):
                    row = 2 * r + u
                    for v in range(H // 16):
                        plsc.addupdate(
                            acc_v.at[row, pl.ds(16 * v, 16)],
                            gath_v[k, row, pl.ds(16 * v, 16)])
                return carry
            lax.fori_loop(0, C // 2, row_body, 0)

        def chunk_body(j, carry):
            pltpu.sync_copy(x_hbm.at[wid, j], idx_v)
            acc_cps = []
            for s in range(NSUB):
                acc_cps.append(pltpu.async_copy(
                    sh_tab.at[idx_v.at[0, s]],
                    acc_v.at[pl.ds(s * 128, 128)],
                    sem_a))
            prev_cps, prev_buf = fire(1, 0), 0
            for cp in acc_cps:
                cp.wait()
            for f in range(2, NF):
                cur_buf = (f - 1) % 2
                cur_cps = fire(f, cur_buf)
                for cp in prev_cps:
                    cp.wait()
                add_from(prev_buf)
                prev_cps, prev_buf = cur_cps, cur_buf
            for cp in prev_cps:
                cp.wait()
            add_from(prev_buf)
            pltpu.sync_copy(acc_v, out_hbm.at[pl.ds(wid * (nchunk * C) + j * C, C)])
            return carry

        lax.fori_loop(0, nchunk, chunk_body, 0)

    return body


def kernel(x, emb_0, emb_1, emb_2, emb_3, emb_4, emb_5, emb_6, emb_7,
           emb_8, emb_9):
    n = x.shape[0]
    rows_per_w = -(-n // (NW * C)) * C
    nchunk = rows_per_w // C
    n_pad = NW * rows_per_w

    xi = x.reshape(n, NF)
    xi = jnp.pad(xi, ((0, n_pad - n), (0, 0)))
    xb = xi.reshape(NW, nchunk, C, NF).transpose(0, 1, 3, 2)
    xb = xb + (jnp.arange(NF, dtype=jnp.int32) * 500).reshape(1, 1, NF, 1)
    xb = xb.reshape(NW, nchunk, NF, NSUB, 128)

    out = _sc_lookup_sum(n_pad, nchunk)(
        xb, emb_0, emb_1, emb_2, emb_3, emb_4, emb_5, emb_6, emb_7,
        emb_8, emb_9)
    return out[:n].reshape(n, 1, H)
